# baseline (speedup 1.0000x reference)
import functools

import jax
import jax.numpy as jnp
import numpy as np
from jax.experimental import pallas as pl
from jax.experimental.pallas import tpu as pltpu


def _gelu(v):
    return 0.5 * v * (1.0 + jax.lax.erf(v * np.float32(0.7071067811865476)))


def _fused_body(x_ref, w1_ref, b1_ref, gw2_ref, gb2_ref, sel_ref, w2_ref,
                o_ref, *, E, H, NL):
    bf16 = jnp.bfloat16
    xt = x_ref[...]
    y = jnp.dot(xt.astype(bf16), w1_ref[...],
                preferred_element_type=jnp.float32) + b1_ref[...]
    h = _gelu(y[:, :H])
    logits = jnp.dot(h.astype(bf16), gw2_ref[...],
                     preferred_element_type=jnp.float32) + gb2_ref[...]
    probs = jax.nn.softmax(logits, axis=-1)
    nl = jnp.concatenate(
        [_gelu(y[:, H:H + NL - 2]),
         jax.nn.sigmoid(y[:, H + NL - 2:H + NL])], axis=1)
    scale = jnp.dot(probs, sel_ref[...],
                    preferred_element_type=jnp.float32)
    z = jnp.concatenate([nl * scale, probs], axis=1)
    o = jnp.dot(z.astype(bf16), w2_ref[...],
                preferred_element_type=jnp.float32)
    o_ref[...] = xt * (1.0 + o[:, :E]) + o[:, E:]


def kernel(x, reglu_u, reglu_a, reglu_b, reglu_bias, film_dw, film_db,
           film_uw, film_ub, p2_w, p2_v, p2_alpha, p2_b, p2_bias, p4_w, p4_v,
           p4_alpha, p4_b, p4_bias, gate_w1, gate_b1, gate_w2, gate_b2,
           expert_bias, post_mix_alpha):
    B, S, E = x.shape
    H = gate_w1.shape[1]
    R = film_dw.shape[-1]
    r2 = p2_w.shape[1]
    r4 = p4_w.shape[1]
    M = gate_w2.shape[1]
    NL = 2 * R + 2 * r2 + 2 * r4 + 2
    K1 = H + NL
    K1P = 128 * ((K1 + 127) // 128)

    f32 = jnp.float32

    w1 = jnp.concatenate([
        gate_w1,
        film_dw[0], film_dw[1],
        p2_w[0].T, p2_w[1].T,
        p4_w[0].T, p4_w[1].T,
        reglu_u[0][:, None], reglu_u[1][:, None],
    ], axis=1)
    w1 = jnp.pad(w1, ((0, 0), (0, K1P - K1)))
    b1 = jnp.concatenate([
        gate_b1,
        film_db[0], film_db[1],
        p2_b[0], p2_b[1],
        p4_b[0], p4_b[1],
        reglu_b[0:1], reglu_b[1:2],
    ])
    b1 = jnp.pad(b1, (0, K1P - K1))[None, :]

    gb2 = (gate_b2 + expert_bias)[None, :]

    sel_np = np.zeros((M, NL), dtype=np.float32)
    c = 0
    sel_np[1, c:c + R] = 1.0; c += R
    sel_np[5, c:c + R] = 1.0; c += R
    sel_np[2, c:c + r2] = 1.0; c += r2
    sel_np[6, c:c + r2] = 1.0; c += r2
    sel_np[3, c:c + r4] = 1.0; c += r4
    sel_np[7, c:c + r4] = 1.0; c += r4
    sel_np[0, c] = 1.0; c += 1
    sel_np[4, c] = 1.0; c += 1
    sel = jnp.asarray(sel_np)

    zE = jnp.zeros((1, E), dtype=f32)
    w2 = jnp.concatenate([
        film_uw[0],
        film_uw[1],
        jnp.concatenate([jnp.zeros((r2, E), f32),
                         p2_alpha[0][:, None] * p2_v[0]], axis=1),
        jnp.concatenate([jnp.zeros((r2, E), f32),
                         p2_alpha[1][:, None] * p2_v[1]], axis=1),
        jnp.concatenate([jnp.zeros((r4, E), f32),
                         p4_alpha[0][:, None] * p4_v[0]], axis=1),
        jnp.concatenate([jnp.zeros((r4, E), f32),
                         p4_alpha[1][:, None] * p4_v[1]], axis=1),
        jnp.concatenate([reglu_a[0][None, :], zE], axis=1),
        jnp.concatenate([reglu_a[1][None, :], zE], axis=1),
        jnp.concatenate([zE, reglu_bias[0][None, :]], axis=1),
        film_ub[0][None, :],
        jnp.concatenate([zE, p2_bias[0][None, :]], axis=1),
        jnp.concatenate([zE, p4_bias[0][None, :]], axis=1),
        jnp.concatenate([zE, reglu_bias[1][None, :]], axis=1),
        film_ub[1][None, :],
        jnp.concatenate([zE, p2_bias[1][None, :]], axis=1),
        jnp.concatenate([zE, p4_bias[1][None, :]], axis=1),
    ], axis=0) * post_mix_alpha

    N = B * S
    T = 512
    x2 = x.reshape(N, E)
    w1 = w1.astype(jnp.bfloat16)
    gw2 = gate_w2.astype(jnp.bfloat16)
    w2 = w2.astype(jnp.bfloat16)

    body = functools.partial(_fused_body, E=E, H=H, NL=NL)
    out = pl.pallas_call(
        body,
        grid=(N // T,),
        in_specs=[
            pl.BlockSpec((T, E), lambda i: (i, 0)),
            pl.BlockSpec((E, K1P), lambda i: (0, 0)),
            pl.BlockSpec((1, K1P), lambda i: (0, 0)),
            pl.BlockSpec((H, M), lambda i: (0, 0)),
            pl.BlockSpec((1, M), lambda i: (0, 0)),
            pl.BlockSpec((M, NL), lambda i: (0, 0)),
            pl.BlockSpec((NL + M, 2 * E), lambda i: (0, 0)),
        ],
        out_specs=pl.BlockSpec((T, E), lambda i: (i, 0)),
        out_shape=jax.ShapeDtypeStruct((N, E), f32),
        compiler_params=pltpu.CompilerParams(
            dimension_semantics=("parallel",)),
    )(x2, w1, b1, gw2, gb2, sel, w2)
    return out.reshape(B, S, E)


# speedup vs baseline: 1.1491x; 1.1491x over previous
import functools

import jax
import jax.numpy as jnp
import numpy as np
from jax.experimental import pallas as pl
from jax.experimental.pallas import tpu as pltpu


def _gelu(v):
    return 0.5 * v * (1.0 + jax.lax.erf(v * np.float32(0.7071067811865476)))


def _fused_body(x_ref, w1_ref, b1_ref, gw2_ref, gb2_ref, sel_ref, w2_ref,
                o_ref, *, E, H, NL):
    bf16 = jnp.bfloat16
    xt = x_ref[...]
    y = jnp.dot(xt.astype(bf16), w1_ref[...],
                preferred_element_type=jnp.float32) + b1_ref[...]
    h = _gelu(y[:, :H])
    logits = jnp.dot(h.astype(bf16), gw2_ref[...],
                     preferred_element_type=jnp.float32) + gb2_ref[...]
    probs = jax.nn.softmax(logits, axis=-1)
    nl = jnp.concatenate(
        [_gelu(y[:, H:H + NL - 2]),
         jax.nn.sigmoid(y[:, H + NL - 2:H + NL])], axis=1)
    scale = jnp.dot(probs, sel_ref[...],
                    preferred_element_type=jnp.float32)
    z = jnp.concatenate([nl * scale, probs], axis=1)
    o = jnp.dot(z.astype(bf16), w2_ref[...],
                preferred_element_type=jnp.float32)
    o_ref[...] = xt * (1.0 + o[:, :E]) + o[:, E:]


def kernel(x, reglu_u, reglu_a, reglu_b, reglu_bias, film_dw, film_db,
           film_uw, film_ub, p2_w, p2_v, p2_alpha, p2_b, p2_bias, p4_w, p4_v,
           p4_alpha, p4_b, p4_bias, gate_w1, gate_b1, gate_w2, gate_b2,
           expert_bias, post_mix_alpha):
    B, S, E = x.shape
    H = gate_w1.shape[1]
    R = film_dw.shape[-1]
    r2 = p2_w.shape[1]
    r4 = p4_w.shape[1]
    M = gate_w2.shape[1]
    NL = 2 * R + 2 * r2 + 2 * r4 + 2
    K1 = H + NL
    K1P = 128 * ((K1 + 127) // 128)

    f32 = jnp.float32

    w1 = jnp.concatenate([
        gate_w1,
        film_dw[0], film_dw[1],
        p2_w[0].T, p2_w[1].T,
        p4_w[0].T, p4_w[1].T,
        reglu_u[0][:, None], reglu_u[1][:, None],
    ], axis=1)
    w1 = jnp.pad(w1, ((0, 0), (0, K1P - K1)))
    b1 = jnp.concatenate([
        gate_b1,
        film_db[0], film_db[1],
        p2_b[0], p2_b[1],
        p4_b[0], p4_b[1],
        reglu_b[0:1], reglu_b[1:2],
    ])
    b1 = jnp.pad(b1, (0, K1P - K1))[None, :]

    gb2 = (gate_b2 + expert_bias)[None, :]

    sel_np = np.zeros((M, NL), dtype=np.float32)
    c = 0
    sel_np[1, c:c + R] = 1.0; c += R
    sel_np[5, c:c + R] = 1.0; c += R
    sel_np[2, c:c + r2] = 1.0; c += r2
    sel_np[6, c:c + r2] = 1.0; c += r2
    sel_np[3, c:c + r4] = 1.0; c += r4
    sel_np[7, c:c + r4] = 1.0; c += r4
    sel_np[0, c] = 1.0; c += 1
    sel_np[4, c] = 1.0; c += 1
    sel = jnp.asarray(sel_np)

    zE = jnp.zeros((1, E), dtype=f32)
    w2 = jnp.concatenate([
        film_uw[0],
        film_uw[1],
        jnp.concatenate([jnp.zeros((r2, E), f32),
                         p2_alpha[0][:, None] * p2_v[0]], axis=1),
        jnp.concatenate([jnp.zeros((r2, E), f32),
                         p2_alpha[1][:, None] * p2_v[1]], axis=1),
        jnp.concatenate([jnp.zeros((r4, E), f32),
                         p4_alpha[0][:, None] * p4_v[0]], axis=1),
        jnp.concatenate([jnp.zeros((r4, E), f32),
                         p4_alpha[1][:, None] * p4_v[1]], axis=1),
        jnp.concatenate([reglu_a[0][None, :], zE], axis=1),
        jnp.concatenate([reglu_a[1][None, :], zE], axis=1),
        jnp.concatenate([zE, reglu_bias[0][None, :]], axis=1),
        film_ub[0][None, :],
        jnp.concatenate([zE, p2_bias[0][None, :]], axis=1),
        jnp.concatenate([zE, p4_bias[0][None, :]], axis=1),
        jnp.concatenate([zE, reglu_bias[1][None, :]], axis=1),
        film_ub[1][None, :],
        jnp.concatenate([zE, p2_bias[1][None, :]], axis=1),
        jnp.concatenate([zE, p4_bias[1][None, :]], axis=1),
    ], axis=0) * post_mix_alpha

    N = B * S
    T = 1024
    x2 = x.reshape(N, E)
    w1 = w1.astype(jnp.bfloat16)
    gw2 = gate_w2.astype(jnp.bfloat16)
    w2 = w2.astype(jnp.bfloat16)

    body = functools.partial(_fused_body, E=E, H=H, NL=NL)
    out = pl.pallas_call(
        body,
        grid=(N // T,),
        in_specs=[
            pl.BlockSpec((T, E), lambda i: (i, 0)),
            pl.BlockSpec((E, K1P), lambda i: (0, 0)),
            pl.BlockSpec((1, K1P), lambda i: (0, 0)),
            pl.BlockSpec((H, M), lambda i: (0, 0)),
            pl.BlockSpec((1, M), lambda i: (0, 0)),
            pl.BlockSpec((M, NL), lambda i: (0, 0)),
            pl.BlockSpec((NL + M, 2 * E), lambda i: (0, 0)),
        ],
        out_specs=pl.BlockSpec((T, E), lambda i: (i, 0)),
        out_shape=jax.ShapeDtypeStruct((N, E), f32),
        compiler_params=pltpu.CompilerParams(
            dimension_semantics=("parallel",)),
    )(x2, w1, b1, gw2, gb2, sel, w2)
    return out.reshape(B, S, E)


# speedup vs baseline: 1.2048x; 1.0484x over previous
import functools

import jax
import jax.numpy as jnp
import numpy as np
from jax.experimental import pallas as pl
from jax.experimental.pallas import tpu as pltpu


def _gelu(v):
    return 0.5 * v * (1.0 + jax.lax.erf(v * np.float32(0.7071067811865476)))


def _fused_body(x_ref, w1_ref, b1_ref, gw2_ref, gb2_ref, sel_ref, w2_ref,
                o_ref, *, E, H, NL):
    bf16 = jnp.bfloat16
    xt = x_ref[...]
    y = jnp.dot(xt.astype(bf16), w1_ref[...],
                preferred_element_type=jnp.float32) + b1_ref[...]
    h = _gelu(y[:, :H])
    logits = jnp.dot(h.astype(bf16), gw2_ref[...],
                     preferred_element_type=jnp.float32) + gb2_ref[...]
    probs = jax.nn.softmax(logits, axis=-1)
    nl = jnp.concatenate(
        [_gelu(y[:, H:H + NL - 2]),
         jax.nn.sigmoid(y[:, H + NL - 2:H + NL])], axis=1)
    scale = jnp.dot(probs, sel_ref[...],
                    preferred_element_type=jnp.float32)
    z = jnp.concatenate([nl * scale, probs], axis=1)
    o = jnp.dot(z.astype(bf16), w2_ref[...],
                preferred_element_type=jnp.float32)
    o_ref[...] = xt * (1.0 + o[:, :E]) + o[:, E:]


def kernel(x, reglu_u, reglu_a, reglu_b, reglu_bias, film_dw, film_db,
           film_uw, film_ub, p2_w, p2_v, p2_alpha, p2_b, p2_bias, p4_w, p4_v,
           p4_alpha, p4_b, p4_bias, gate_w1, gate_b1, gate_w2, gate_b2,
           expert_bias, post_mix_alpha):
    B, S, E = x.shape
    H = gate_w1.shape[1]
    R = film_dw.shape[-1]
    r2 = p2_w.shape[1]
    r4 = p4_w.shape[1]
    M = gate_w2.shape[1]
    NL = 2 * R + 2 * r2 + 2 * r4 + 2
    K1 = H + NL
    K1P = 128 * ((K1 + 127) // 128)

    f32 = jnp.float32

    w1 = jnp.concatenate([
        gate_w1,
        film_dw[0], film_dw[1],
        p2_w[0].T, p2_w[1].T,
        p4_w[0].T, p4_w[1].T,
        reglu_u[0][:, None], reglu_u[1][:, None],
    ], axis=1)
    w1 = jnp.pad(w1, ((0, 0), (0, K1P - K1)))
    b1 = jnp.concatenate([
        gate_b1,
        film_db[0], film_db[1],
        p2_b[0], p2_b[1],
        p4_b[0], p4_b[1],
        reglu_b[0:1], reglu_b[1:2],
    ])
    b1 = jnp.pad(b1, (0, K1P - K1))[None, :]

    gb2 = (gate_b2 + expert_bias)[None, :]

    sel_np = np.zeros((M, NL), dtype=np.float32)
    c = 0
    sel_np[1, c:c + R] = 1.0; c += R
    sel_np[5, c:c + R] = 1.0; c += R
    sel_np[2, c:c + r2] = 1.0; c += r2
    sel_np[6, c:c + r2] = 1.0; c += r2
    sel_np[3, c:c + r4] = 1.0; c += r4
    sel_np[7, c:c + r4] = 1.0; c += r4
    sel_np[0, c] = 1.0; c += 1
    sel_np[4, c] = 1.0; c += 1
    sel = jnp.asarray(sel_np)

    zE = jnp.zeros((1, E), dtype=f32)
    w2 = jnp.concatenate([
        film_uw[0],
        film_uw[1],
        jnp.concatenate([jnp.zeros((r2, E), f32),
                         p2_alpha[0][:, None] * p2_v[0]], axis=1),
        jnp.concatenate([jnp.zeros((r2, E), f32),
                         p2_alpha[1][:, None] * p2_v[1]], axis=1),
        jnp.concatenate([jnp.zeros((r4, E), f32),
                         p4_alpha[0][:, None] * p4_v[0]], axis=1),
        jnp.concatenate([jnp.zeros((r4, E), f32),
                         p4_alpha[1][:, None] * p4_v[1]], axis=1),
        jnp.concatenate([reglu_a[0][None, :], zE], axis=1),
        jnp.concatenate([reglu_a[1][None, :], zE], axis=1),
        jnp.concatenate([zE, reglu_bias[0][None, :]], axis=1),
        film_ub[0][None, :],
        jnp.concatenate([zE, p2_bias[0][None, :]], axis=1),
        jnp.concatenate([zE, p4_bias[0][None, :]], axis=1),
        jnp.concatenate([zE, reglu_bias[1][None, :]], axis=1),
        film_ub[1][None, :],
        jnp.concatenate([zE, p2_bias[1][None, :]], axis=1),
        jnp.concatenate([zE, p4_bias[1][None, :]], axis=1),
    ], axis=0) * post_mix_alpha

    N = B * S
    T = 2048
    x2 = x.reshape(N, E)
    w1 = w1.astype(jnp.bfloat16)
    gw2 = gate_w2.astype(jnp.bfloat16)
    w2 = w2.astype(jnp.bfloat16)

    body = functools.partial(_fused_body, E=E, H=H, NL=NL)
    out = pl.pallas_call(
        body,
        grid=(N // T,),
        in_specs=[
            pl.BlockSpec((T, E), lambda i: (i, 0)),
            pl.BlockSpec((E, K1P), lambda i: (0, 0)),
            pl.BlockSpec((1, K1P), lambda i: (0, 0)),
            pl.BlockSpec((H, M), lambda i: (0, 0)),
            pl.BlockSpec((1, M), lambda i: (0, 0)),
            pl.BlockSpec((M, NL), lambda i: (0, 0)),
            pl.BlockSpec((NL + M, 2 * E), lambda i: (0, 0)),
        ],
        out_specs=pl.BlockSpec((T, E), lambda i: (i, 0)),
        out_shape=jax.ShapeDtypeStruct((N, E), f32),
        compiler_params=pltpu.CompilerParams(
            dimension_semantics=("parallel",)),
    )(x2, w1, b1, gw2, gb2, sel, w2)
    return out.reshape(B, S, E)
